# baseline (speedup 1.0000x reference)
import functools

import jax
import jax.numpy as jnp
from jax import lax
from jax.experimental import pallas as pl
from jax.experimental.pallas import tpu as pltpu
from jax.experimental.pallas import tpu_sc as plsc

_BATCH = 16384
_DIM = 2048

_CHUNK_ROWS = 16


@functools.lru_cache(maxsize=1)
def _make_sc_copy():
    info = plsc.get_sparse_core_info()
    nw = info.num_cores * info.num_subcores
    rows_per_tile = _BATCH // nw
    nc = info.num_cores
    nchunks = rows_per_tile // _CHUNK_ROWS
    mesh = plsc.VectorSubcoreMesh(core_axis_name="c", subcore_axis_name="s")

    @functools.partial(
        pl.kernel,
        mesh=mesh,
        out_type=jax.ShapeDtypeStruct((_BATCH, _DIM), jnp.float32),
        scratch_types=[
            pltpu.VMEM_SHARED((16, 2, _CHUNK_ROWS, _DIM), jnp.float32),
            pltpu.VMEM((2, _CHUNK_ROWS, _DIM), jnp.float32),
            pltpu.SemaphoreType.DMA((4,)),
            pltpu.SemaphoreType.DMA((4,)),
        ],
    )
    def sc_copy(h_hbm, out_hbm, shared, tbuf, rsem, wsem):
        sid = lax.axis_index("s")
        wid = sid * nc + lax.axis_index("c")
        base = wid * rows_per_tile
        bufs = (shared.at[sid, 0], tbuf.at[0], shared.at[sid, 1],
                tbuf.at[1])
        depth = 4

        def rd(i, b):
            return pltpu.make_async_copy(
                h_hbm.at[pl.ds(base + i * _CHUNK_ROWS, _CHUNK_ROWS), :],
                bufs[b], rsem.at[b])

        def wr(i, b):
            return pltpu.make_async_copy(
                bufs[b],
                out_hbm.at[pl.ds(base + i * _CHUNK_ROWS, _CHUNK_ROWS), :],
                wsem.at[b])

        for j in range(depth - 1):
            rd(j, j).start()
        for i in range(nchunks):
            b = i % depth
            nxt = (i + depth - 1) % depth
            if i >= 1:
                wr(i - 1, (i - 1) % depth).wait()
            if i + depth - 1 < nchunks:
                rd(i + depth - 1, nxt).start()
            rd(i, b).wait()
            wr(i, b).start()
        wr(nchunks - 1, (nchunks - 1) % depth).wait()

    return sc_copy


def kernel(h):
    return _make_sc_copy()(h)


# speedup vs baseline: 1.0247x; 1.0247x over previous
import functools

import jax
import jax.numpy as jnp
from jax import lax
from jax.experimental import pallas as pl
from jax.experimental.pallas import tpu as pltpu
from jax.experimental.pallas import tpu_sc as plsc

_BATCH = 16384
_DIM = 2048
_TC_ROWS = 8192
_TC_BLOCK_ROWS = 1024
_CHUNK_ROWS = 16


@functools.lru_cache(maxsize=1)
def _make_sc_copy():
    info = plsc.get_sparse_core_info()
    nw = info.num_cores * info.num_subcores
    nc = info.num_cores
    sc_rows = _BATCH - _TC_ROWS
    rows_per_tile = sc_rows // nw
    nchunks = rows_per_tile // _CHUNK_ROWS
    depth = 3
    mesh = plsc.VectorSubcoreMesh(core_axis_name="c", subcore_axis_name="s")

    @functools.partial(
        pl.kernel,
        mesh=mesh,
        out_type=jax.ShapeDtypeStruct((_BATCH, _DIM), jnp.float32),
        scratch_types=[
            pltpu.VMEM_SHARED((16, depth, _CHUNK_ROWS, _DIM), jnp.float32),
            pltpu.SemaphoreType.DMA((depth,)),
            pltpu.SemaphoreType.DMA((depth,)),
        ],
    )
    def sc_copy(h_hbm, out_hbm, shared, rsem, wsem):
        sid = lax.axis_index("s")
        wid = sid * nc + lax.axis_index("c")
        base = _TC_ROWS + wid * rows_per_tile
        bufs = tuple(shared.at[sid, j] for j in range(depth))

        def rd(i, b):
            return pltpu.make_async_copy(
                h_hbm.at[pl.ds(base + i * _CHUNK_ROWS, _CHUNK_ROWS), :],
                bufs[b], rsem.at[b])

        def wr(i, b):
            return pltpu.make_async_copy(
                bufs[b],
                out_hbm.at[pl.ds(base + i * _CHUNK_ROWS, _CHUNK_ROWS), :],
                wsem.at[b])

        for j in range(depth - 1):
            rd(j, j).start()
        for i in range(nchunks):
            b = i % depth
            if i >= 1:
                wr(i - 1, (i - 1) % depth).wait()
            if i + depth - 1 < nchunks:
                rd(i + depth - 1, (i + depth - 1) % depth).start()
            rd(i, b).wait()
            wr(i, b).start()
        wr(nchunks - 1, (nchunks - 1) % depth).wait()

    return sc_copy


def _tc_copy_body(h_ref, _, o_ref):
    o_ref[...] = h_ref[...]


def kernel(h):
    partial = _make_sc_copy()(h)
    return pl.pallas_call(
        _tc_copy_body,
        grid=(_TC_ROWS // _TC_BLOCK_ROWS,),
        in_specs=[
            pl.BlockSpec((_TC_BLOCK_ROWS, _DIM), lambda i: (i, 0)),
            pl.BlockSpec(memory_space=pl.ANY),
        ],
        out_specs=pl.BlockSpec((_TC_BLOCK_ROWS, _DIM), lambda i: (i, 0)),
        out_shape=jax.ShapeDtypeStruct((_BATCH, _DIM), jnp.float32),
        input_output_aliases={1: 0},
    )(h, partial)
